# baseline (device time: 12971 ns/iter reference)
import jax
import jax.numpy as jnp
from jax import lax
from jax.experimental import pallas as pl
from jax.experimental.pallas import tpu as pltpu

N_DEV = 4
GRID = 8


def kernel(x):
    m_per, n = x.shape
    m_global = N_DEV * m_per
    n_blk = n // GRID

    def body(x_ref, out_ref, comm_ref, send_sems, recv_sems):
        step = pl.program_id(0)
        my = lax.axis_index("i")

        @pl.when(step == 0)
        def _():
            barrier_sem = pltpu.get_barrier_semaphore()
            for d in range(1, N_DEV):
                pl.semaphore_signal(
                    barrier_sem, inc=1,
                    device_id=((my + d) % N_DEV,),
                    device_id_type=pl.DeviceIdType.MESH,
                )
            pl.semaphore_wait(barrier_sem, N_DEV - 1)

        cols = pl.ds(step * n_blk, n_blk)
        comm_ref[0, :, cols] = jnp.sum(x_ref[:, :], axis=0, keepdims=True)

        for d in range(1, N_DEV):
            pltpu.make_async_remote_copy(
                src_ref=comm_ref.at[0, :, cols],
                dst_ref=comm_ref.at[d, :, cols],
                send_sem=send_sems.at[d - 1, step],
                recv_sem=recv_sems.at[d - 1, step],
                device_id=((my + d) % N_DEV,),
                device_id_type=pl.DeviceIdType.MESH,
            ).start()

        @pl.when(step == GRID - 1)
        def _():
            for d in range(1, N_DEV):
                for s in range(GRID):
                    scols = pl.ds(s * n_blk, n_blk)
                    w = pltpu.make_async_remote_copy(
                        src_ref=comm_ref.at[0, :, scols],
                        dst_ref=comm_ref.at[d, :, scols],
                        send_sem=send_sems.at[d - 1, s],
                        recv_sem=recv_sems.at[d - 1, s],
                        device_id=((my + d) % N_DEV,),
                        device_id_type=pl.DeviceIdType.MESH,
                    )
                    w.wait_send()
                    w.wait_recv()

            acc = comm_ref[0, :, :]
            for d in range(1, N_DEV):
                acc = acc + comm_ref[d, :, :]
            out_ref[:, :] = acc * (1.0 / m_global)

    return pl.pallas_call(
        body,
        grid=(GRID,),
        out_shape=jax.ShapeDtypeStruct((1, n), jnp.float32),
        in_specs=[pl.BlockSpec((m_per, n_blk), lambda i: (0, i))],
        out_specs=pl.BlockSpec((1, n), lambda i: (0, 0)),
        scratch_shapes=[
            pltpu.VMEM((N_DEV, 1, n), jnp.float32),
            pltpu.SemaphoreType.DMA((N_DEV - 1, GRID)),
            pltpu.SemaphoreType.DMA((N_DEV - 1, GRID)),
        ],
        compiler_params=pltpu.CompilerParams(collective_id=0),
    )(x)


# device time: 10066 ns/iter; 1.2886x vs baseline; 1.2886x over previous
import jax
import jax.numpy as jnp
from jax import lax
from jax.experimental import pallas as pl
from jax.experimental.pallas import tpu as pltpu

N_DEV = 4
NBLK = 8


def kernel(x):
    m_per, n = x.shape
    m_global = N_DEV * m_per
    m_blk = m_per // NBLK

    def body(x_hbm, out_ref, buf_ref, load_sems, comm_ref, send_sems, recv_sems):
        my = lax.axis_index("i")

        def load(b):
            return pltpu.make_async_copy(
                x_hbm.at[pl.ds(b * m_blk, m_blk), :],
                buf_ref.at[b % 2],
                load_sems.at[b % 2],
            )

        load(0).start()

        barrier_sem = pltpu.get_barrier_semaphore()
        for d in range(1, N_DEV):
            pl.semaphore_signal(
                barrier_sem, inc=1,
                device_id=((my + d) % N_DEV,),
                device_id_type=pl.DeviceIdType.MESH,
            )
        pl.semaphore_wait(barrier_sem, N_DEV - 1)

        acc = jnp.zeros((1, n), jnp.float32)
        for b in range(NBLK):
            if b + 1 < NBLK:
                load(b + 1).start()
            load(b).wait()
            acc = acc + jnp.sum(buf_ref[b % 2], axis=0, keepdims=True)

        comm_ref[0, :, :] = acc
        rdmas = []
        for d in range(1, N_DEV):
            rdma = pltpu.make_async_remote_copy(
                src_ref=comm_ref.at[0],
                dst_ref=comm_ref.at[d],
                send_sem=send_sems.at[d - 1],
                recv_sem=recv_sems.at[d - 1],
                device_id=((my + d) % N_DEV,),
                device_id_type=pl.DeviceIdType.MESH,
            )
            rdma.start()
            rdmas.append(rdma)
        for rdma in rdmas:
            rdma.wait()

        total = comm_ref[0, :, :]
        for d in range(1, N_DEV):
            total = total + comm_ref[d, :, :]
        out_ref[:, :] = total * (1.0 / m_global)

    return pl.pallas_call(
        body,
        out_shape=jax.ShapeDtypeStruct((1, n), jnp.float32),
        in_specs=[pl.BlockSpec(memory_space=pl.ANY)],
        out_specs=pl.BlockSpec(memory_space=pltpu.VMEM),
        scratch_shapes=[
            pltpu.VMEM((2, m_blk, n), jnp.float32),
            pltpu.SemaphoreType.DMA((2,)),
            pltpu.VMEM((N_DEV, 1, n), jnp.float32),
            pltpu.SemaphoreType.DMA((N_DEV - 1,)),
            pltpu.SemaphoreType.DMA((N_DEV - 1,)),
        ],
        compiler_params=pltpu.CompilerParams(collective_id=0),
    )(x)
